# baseline (speedup 1.0000x reference)
import functools

import jax
import jax.numpy as jnp
from jax import lax
from jax.experimental import pallas as pl
from jax.experimental.pallas import tpu as pltpu
from jax.experimental.pallas import tpu_sc as plsc

B = 4096
L = 200
H = 64
N = B * L
CHUNK = 128
LANES = 16


@functools.lru_cache(maxsize=None)
def _build(nc: int, ns: int):
    nw = nc * ns
    per_w = N // nw
    n_chunks = per_w // CHUNK
    assert per_w % CHUNK == 0 and n_chunks % 2 == 0 and per_w % L == 0

    mesh = plsc.VectorSubcoreMesh(
        core_axis_name="c", subcore_axis_name="s",
        num_cores=nc, num_subcores=ns,
    )

    @functools.partial(
        pl.kernel,
        out_type=jax.ShapeDtypeStruct((N, H), jnp.float32),
        mesh=mesh,
        compiler_params=pltpu.CompilerParams(use_tc_tiling_on_sc=False),
        scratch_types=[
            pltpu.VMEM((n_chunks, CHUNK), jnp.int32),
            pltpu.VMEM((2 * L, H), jnp.float32),
            pltpu.VMEM((CHUNK, H), jnp.float32),
            pltpu.VMEM((CHUNK, H), jnp.float32),
            pltpu.SemaphoreType.DMA,
            pltpu.SemaphoreType.DMA,
            pltpu.SemaphoreType.DMA,
            pltpu.SemaphoreType.DMA,
        ],
    )
    def run(idx_hbm, table_hbm, pos2_hbm, out_hbm,
            idx_v, pos_v, buf0, buf1, g0, g1, s0, s1):
        wid = lax.axis_index("s") * nc + lax.axis_index("c")
        row0 = wid * per_w

        pltpu.sync_copy(idx_hbm.at[pl.ds(wid * n_chunks, n_chunks)], idx_v)
        pltpu.sync_copy(pos2_hbm, pos_v)

        def start_gather(c, buf, sem):
            pltpu.async_copy(table_hbm.at[idx_v.at[c]], buf, sem)

        def wait_gather(c, buf, sem):
            pltpu.make_async_copy(table_hbm.at[idx_v.at[c]], buf, sem).wait()

        def start_scatter(c, buf, sem):
            pltpu.async_copy(buf, out_hbm.at[pl.ds(row0 + c * CHUNK, CHUNK)], sem)

        def wait_scatter(c, buf, sem):
            pltpu.make_async_copy(
                buf, out_hbm.at[pl.ds(row0 + c * CHUNK, CHUNK)], sem).wait()

        def add_pos(c, buf):
            ph = lax.rem(c * CHUNK, L)

            def rbody(r, _):
                pr = ph + r
                for q in range(H // LANES):
                    sl = pl.ds(q * LANES, LANES)
                    buf[r, sl] = buf[r, sl] + pos_v[pr, sl]
                return 0

            lax.fori_loop(0, CHUNK, rbody, 0, unroll=4)

        start_gather(0, buf0, g0)

        def cbody(cc, _):
            a = 2 * cc
            b = a + 1

            @pl.when(cc > 0)
            def _():
                wait_scatter(b - 2, buf1, s1)
            start_gather(b, buf1, g1)

            wait_gather(a, buf0, g0)
            add_pos(a, buf0)
            start_scatter(a, buf0, s0)

            wait_scatter(a, buf0, s0)
            @pl.when(cc < n_chunks // 2 - 1)
            def _():
                start_gather(a + 2, buf0, g0)

            wait_gather(b, buf1, g1)
            add_pos(b, buf1)
            start_scatter(b, buf1, s1)
            return 0

        lax.fori_loop(0, n_chunks // 2, cbody, 0)
        wait_scatter(n_chunks - 1, buf1, s1)

    return run


def kernel(input_ids, embed_table, pos_embed):
    info = plsc.get_sparse_core_info()
    run = _build(info.num_cores, info.num_subcores)
    idx = jnp.reshape(input_ids.astype(jnp.int32), (N // CHUNK, CHUNK))
    pos2 = jnp.concatenate([pos_embed, pos_embed], axis=0)
    out = run(idx, embed_table, pos2)
    return jnp.reshape(out, (B, L, H))


# speedup vs baseline: 1.0031x; 1.0031x over previous
import functools

import jax
import jax.numpy as jnp
from jax import lax
from jax.experimental import pallas as pl
from jax.experimental.pallas import tpu as pltpu
from jax.experimental.pallas import tpu_sc as plsc

B = 4096
L = 200
H = 64
N = B * L
CHUNK = 128
LANES = 16


SUB = 4
SROWS = SUB * CHUNK


@functools.lru_cache(maxsize=None)
def _build(nc: int, ns: int):
    nw = nc * ns
    per_w = N // nw
    n_chunks = per_w // CHUNK
    n_super = per_w // SROWS
    assert per_w % SROWS == 0 and per_w % L == 0

    mesh = plsc.VectorSubcoreMesh(
        core_axis_name="c", subcore_axis_name="s",
        num_cores=nc, num_subcores=ns,
    )

    @functools.partial(
        pl.kernel,
        out_type=jax.ShapeDtypeStruct((N, H), jnp.float32),
        mesh=mesh,
        compiler_params=pltpu.CompilerParams(use_tc_tiling_on_sc=False),
        scratch_types=[
            pltpu.VMEM((n_chunks, CHUNK), jnp.int32),
            pltpu.VMEM((2 * L, H), jnp.float32),
            pltpu.VMEM((SROWS, H), jnp.float32),
            pltpu.VMEM((SROWS, H), jnp.float32),
            pltpu.SemaphoreType.DMA,
            pltpu.SemaphoreType.DMA,
            pltpu.SemaphoreType.DMA,
            pltpu.SemaphoreType.DMA,
        ],
    )
    def run(idx_hbm, table_hbm, pos2_hbm, out_hbm,
            idx_v, pos_v, buf0, buf1, g0, g1, s0, s1):
        wid = lax.axis_index("s") * nc + lax.axis_index("c")
        row0 = wid * per_w

        pltpu.sync_copy(idx_hbm.at[pl.ds(wid * n_chunks, n_chunks)], idx_v)
        pltpu.sync_copy(pos2_hbm, pos_v)

        def start_gathers(sc, buf, sem):
            for j in range(SUB):
                pltpu.async_copy(
                    table_hbm.at[idx_v.at[sc * SUB + j]],
                    buf.at[pl.ds(j * CHUNK, CHUNK)], sem)

        def wait_gathers(sc, buf, sem):
            for j in range(SUB):
                pltpu.make_async_copy(
                    table_hbm.at[idx_v.at[sc * SUB + j]],
                    buf.at[pl.ds(j * CHUNK, CHUNK)], sem).wait()

        def start_scatter(sc, buf, sem):
            pltpu.async_copy(buf, out_hbm.at[pl.ds(row0 + sc * SROWS, SROWS)], sem)

        def wait_scatter(sc, buf, sem):
            pltpu.make_async_copy(
                buf, out_hbm.at[pl.ds(row0 + sc * SROWS, SROWS)], sem).wait()

        def add_pos(sc, buf):
            for j in range(SUB):
                ph = lax.rem((sc * SUB + j) * CHUNK, L)

                def rbody(r, _, j=j, ph=ph):
                    for q in range(H // LANES):
                        sl = pl.ds(q * LANES, LANES)
                        buf[j * CHUNK + r, sl] = (
                            buf[j * CHUNK + r, sl] + pos_v[ph + r, sl])
                    return 0

                lax.fori_loop(0, CHUNK, rbody, 0, unroll=8)

        start_gathers(0, buf0, g0)

        def sbody(s, _):
            even = lax.rem(s, 2) == 0

            def one(cur, gcur, scur, other, gother, sother):
                wait_gathers(s, cur, gcur)

                @pl.when(s + 1 < n_super)
                def _():
                    @pl.when(s >= 1)
                    def _():
                        wait_scatter(s - 1, other, sother)
                    start_gathers(s + 1, other, gother)

                add_pos(s, cur)
                start_scatter(s, cur, scur)

            @pl.when(even)
            def _():
                one(buf0, g0, s0, buf1, g1, s1)

            @pl.when(jnp.logical_not(even))
            def _():
                one(buf1, g1, s1, buf0, g0, s0)
            return 0

        lax.fori_loop(0, n_super, sbody, 0)
        wait_scatter(n_super - 2, buf0 if n_super % 2 == 0 else buf1,
                     s0 if n_super % 2 == 0 else s1)
        wait_scatter(n_super - 1, buf1 if n_super % 2 == 0 else buf0,
                     s1 if n_super % 2 == 0 else s0)

    return run


def kernel(input_ids, embed_table, pos_embed):
    info = plsc.get_sparse_core_info()
    run = _build(info.num_cores, info.num_subcores)
    idx = jnp.reshape(input_ids.astype(jnp.int32), (N // CHUNK, CHUNK))
    pos2 = jnp.concatenate([pos_embed, pos_embed], axis=0)
    out = run(idx, embed_table, pos2)
    return jnp.reshape(out, (B, L, H))


# speedup vs baseline: 1.2781x; 1.2742x over previous
import functools

import jax
import jax.numpy as jnp
from jax import lax
from jax.experimental import pallas as pl
from jax.experimental.pallas import tpu as pltpu
from jax.experimental.pallas import tpu_sc as plsc

B = 4096
L = 200
H = 64
N = B * L
CHUNK = 128
LANES = 16


SUB = 4
SROWS = SUB * CHUNK


@functools.lru_cache(maxsize=None)
def _build(nc: int, ns: int):
    nw = nc * ns
    per_w = N // nw
    n_chunks = per_w // CHUNK
    n_super = per_w // SROWS
    assert per_w % SROWS == 0 and per_w % L == 0

    mesh = plsc.VectorSubcoreMesh(
        core_axis_name="c", subcore_axis_name="s",
        num_cores=nc, num_subcores=ns,
    )

    @functools.partial(
        pl.kernel,
        out_type=jax.ShapeDtypeStruct((N, H), jnp.float32),
        mesh=mesh,
        compiler_params=pltpu.CompilerParams(use_tc_tiling_on_sc=False),
        scratch_types=[
            pltpu.VMEM((n_chunks, CHUNK), jnp.int32),
            pltpu.VMEM((2 * L, H), jnp.float32),
            pltpu.VMEM((SROWS, H), jnp.float32),
            pltpu.VMEM((SROWS, H), jnp.float32),
            pltpu.SemaphoreType.DMA,
            pltpu.SemaphoreType.DMA,
            pltpu.SemaphoreType.DMA,
            pltpu.SemaphoreType.DMA,
        ],
    )
    def run(idx_hbm, table_hbm, pos2_hbm, out_hbm,
            idx_v, pos_v, buf0, buf1, g0, g1, s0, s1):
        wid = lax.axis_index("s") * nc + lax.axis_index("c")
        row0 = wid * per_w

        pltpu.sync_copy(idx_hbm.at[pl.ds(wid * n_chunks, n_chunks)], idx_v)
        pltpu.sync_copy(pos2_hbm, pos_v)

        def start_gathers(sc, buf, sem):
            for j in range(SUB):
                pltpu.async_copy(
                    table_hbm.at[idx_v.at[sc * SUB + j]],
                    buf.at[pl.ds(j * CHUNK, CHUNK)], sem)

        def wait_gathers(sc, buf, sem):
            for j in range(SUB):
                pltpu.make_async_copy(
                    table_hbm.at[idx_v.at[sc * SUB + j]],
                    buf.at[pl.ds(j * CHUNK, CHUNK)], sem).wait()

        def start_scatter(sc, buf, sem):
            pltpu.async_copy(buf, out_hbm.at[pl.ds(row0 + sc * SROWS, SROWS)], sem)

        def wait_scatter(sc, buf, sem):
            pltpu.make_async_copy(
                buf, out_hbm.at[pl.ds(row0 + sc * SROWS, SROWS)], sem).wait()

        def add_pos(sc, buf):
            for j in range(SUB):
                ph = lax.rem((sc * SUB + j) * CHUNK, L)

                @plsc.parallel_loop(0, CHUNK, step=1, unroll=8)
                def _(r, j=j, ph=ph):
                    for q in range(H // LANES):
                        sl = pl.ds(q * LANES, LANES)
                        buf[j * CHUNK + r, sl] = (
                            buf[j * CHUNK + r, sl] + pos_v[ph + r, sl])

        start_gathers(0, buf0, g0)

        def sbody(s, _):
            even = lax.rem(s, 2) == 0

            def one(cur, gcur, scur, other, gother, sother):
                wait_gathers(s, cur, gcur)

                @pl.when(s + 1 < n_super)
                def _():
                    @pl.when(s >= 1)
                    def _():
                        wait_scatter(s - 1, other, sother)
                    start_gathers(s + 1, other, gother)

                add_pos(s, cur)
                start_scatter(s, cur, scur)

            @pl.when(even)
            def _():
                one(buf0, g0, s0, buf1, g1, s1)

            @pl.when(jnp.logical_not(even))
            def _():
                one(buf1, g1, s1, buf0, g0, s0)
            return 0

        lax.fori_loop(0, n_super, sbody, 0)
        wait_scatter(n_super - 2, buf0 if n_super % 2 == 0 else buf1,
                     s0 if n_super % 2 == 0 else s1)
        wait_scatter(n_super - 1, buf1 if n_super % 2 == 0 else buf0,
                     s1 if n_super % 2 == 0 else s0)

    return run


def kernel(input_ids, embed_table, pos_embed):
    info = plsc.get_sparse_core_info()
    run = _build(info.num_cores, info.num_subcores)
    idx = jnp.reshape(input_ids.astype(jnp.int32), (N // CHUNK, CHUNK))
    pos2 = jnp.concatenate([pos_embed, pos_embed], axis=0)
    out = run(idx, embed_table, pos2)
    return jnp.reshape(out, (B, L, H))
